# Initial kernel scaffold; baseline (speedup 1.0000x reference)
import jax
import jax.numpy as jnp
from jax.experimental import pallas as pl


def kernel(x, Wred, wg, W1, b1, W2, b2):
    raise NotImplementedError("write your pallas kernel here")


# speedup vs baseline: 2.4908x; 2.4908x over previous
import functools

import jax
import jax.numpy as jnp
from jax import lax
from jax.experimental import pallas as pl
from jax.experimental.pallas import tpu as pltpu
from jax.experimental.pallas import tpu_sc as plsc

B, T, C, H, E = 1, 2048, 768, 3072, 8
TB = 128
NB = T // TB + E
NPAD = NB * TB
NW = 32
BPW = NPAD // NW


def _gating_body(x_ref, wred_ref, wg_ref, score_ref, idx_ref):
    xf = x_ref[...]
    red = jnp.dot(xf, wred_ref[...].T,
                  preferred_element_type=jnp.float32)
    wg = wg_ref[...]
    norm = jnp.sqrt(jnp.sum(wg * wg, axis=1, keepdims=True))
    wg_s = wg * (1.5 / norm)
    n2 = jnp.sqrt(jnp.sum(wg_s * wg_s, axis=1, keepdims=True))
    wg_n = wg_s / jnp.maximum(n2, 1e-4)
    logits = jnp.dot(red, wg_n.T,
                     preferred_element_type=jnp.float32)
    lmax = jnp.max(logits, axis=1, keepdims=True)
    z = jnp.sum(jnp.exp(logits - lmax), axis=1, keepdims=True)
    score_ref[...] = 1.0 / z
    col = lax.broadcasted_iota(jnp.int32, logits.shape, 1)
    idx_ref[...] = jnp.min(
        jnp.where(logits >= lmax, col, jnp.int32(E)), axis=1, keepdims=True
    )


def _gating_tc(xf, Wred, wg):
    return pl.pallas_call(
        _gating_body,
        out_shape=(
            jax.ShapeDtypeStruct((T, 1), jnp.float32),
            jax.ShapeDtypeStruct((T, 1), jnp.int32),
        ),
    )(xf, Wred, wg)


def _sc_gather_body(x_hbm, src_hbm, out_hbm, idx_v, rows_v, sem):
    wid = lax.axis_index("s") * 2 + lax.axis_index("c")
    base = wid * BPW
    pltpu.sync_copy(src_hbm.at[pl.ds(base, BPW)], idx_v)
    pltpu.async_copy(x_hbm.at[idx_v], rows_v, sem).wait()
    pltpu.sync_copy(rows_v, out_hbm.at[pl.ds(base, BPW)])


def _sc_gather(xf, src_ids):
    mesh = plsc.VectorSubcoreMesh(core_axis_name="c", subcore_axis_name="s")
    return pl.kernel(
        _sc_gather_body,
        out_type=jax.ShapeDtypeStruct((NPAD, C), jnp.float32),
        mesh=mesh,
        scratch_types=[
            pltpu.VMEM((BPW,), jnp.int32),
            pltpu.VMEM((BPW, C), jnp.float32),
            pltpu.SemaphoreType.DMA,
        ],
    )(xf, src_ids)


def _sc_scatter_body(rows_hbm, dst_hbm, out_hbm, idx_v, rows_v, sem):
    wid = lax.axis_index("s") * 2 + lax.axis_index("c")
    base = wid * BPW
    pltpu.sync_copy(dst_hbm.at[pl.ds(base, BPW)], idx_v)
    pltpu.sync_copy(rows_hbm.at[pl.ds(base, BPW)], rows_v)
    pltpu.async_copy(rows_v, out_hbm.at[idx_v], sem).wait()


def _sc_scatter(rows, dst_ids):
    mesh = plsc.VectorSubcoreMesh(core_axis_name="c", subcore_axis_name="s")
    return pl.kernel(
        _sc_scatter_body,
        out_type=jax.ShapeDtypeStruct((T + 8, C), jnp.float32),
        mesh=mesh,
        scratch_types=[
            pltpu.VMEM((BPW,), jnp.int32),
            pltpu.VMEM((BPW, C), jnp.float32),
            pltpu.SemaphoreType.DMA,
        ],
    )(rows, dst_ids)


def _ffn_body(ex_ref, xs_ref, sc_ref, w1_ref, b1_ref, w2_ref, b2_ref,
              ys_ref, sum_ref):
    u = pl.program_id(0)
    xt = xs_ref[...]
    h = jnp.dot(xt, w1_ref[0], preferred_element_type=jnp.float32)
    h = h + b1_ref[0]
    h = 0.5 * h * (1.0 + lax.erf(h * 0.7071067811865476))
    y = jnp.dot(h, w2_ref[0], preferred_element_type=jnp.float32)
    y = (y + b2_ref[0]) * sc_ref[...]
    ys_ref[...] = y

    @pl.when(u == 0)
    def _():
        sum_ref[...] = jnp.zeros((1, 1), jnp.float32)

    sum_ref[...] += jnp.sum(y).reshape(1, 1)


def _ffn_tc(xs, scores_pad, unit_expert, W1, b1, W2, b2):
    grid_spec = pltpu.PrefetchScalarGridSpec(
        num_scalar_prefetch=1,
        grid=(NB,),
        in_specs=[
            pl.BlockSpec((TB, C), lambda u, ex: (u, 0)),
            pl.BlockSpec((TB, 1), lambda u, ex: (u, 0)),
            pl.BlockSpec((1, C, H), lambda u, ex: (ex[u], 0, 0)),
            pl.BlockSpec((1, 1, H), lambda u, ex: (ex[u], 0, 0)),
            pl.BlockSpec((1, H, C), lambda u, ex: (ex[u], 0, 0)),
            pl.BlockSpec((1, 1, C), lambda u, ex: (ex[u], 0, 0)),
        ],
        out_specs=[
            pl.BlockSpec((TB, C), lambda u, ex: (u, 0)),
            pl.BlockSpec((1, 1), lambda u, ex: (0, 0)),
        ],
    )
    return pl.pallas_call(
        _ffn_body,
        grid_spec=grid_spec,
        out_shape=(
            jax.ShapeDtypeStruct((NPAD, C), jnp.float32),
            jax.ShapeDtypeStruct((1, 1), jnp.float32),
        ),
    )(unit_expert, xs, scores_pad,
      W1, b1.reshape(E, 1, H), W2, b2.reshape(E, 1, C))


def _route(idx):
    onehot = (idx[:, None] == jnp.arange(E, dtype=jnp.int32)[None, :])
    onehot = onehot.astype(jnp.int32)
    ranks_all = jnp.cumsum(onehot, axis=0)
    counts = ranks_all[-1]
    rank = jnp.take_along_axis(ranks_all, idx[:, None], axis=1)[:, 0] - 1
    tiles = (counts + TB - 1) // TB
    tile_off = jnp.concatenate([jnp.zeros((1,), jnp.int32),
                                jnp.cumsum(tiles)]).astype(jnp.int32)
    pos = tile_off[idx] * TB + rank
    tok = jnp.arange(T, dtype=jnp.int32)
    src_ids = jnp.zeros((NPAD,), jnp.int32).at[pos].set(tok)
    dst_ids = jnp.full((NPAD,), T, jnp.int32).at[pos].set(tok)
    u = jnp.arange(NB, dtype=jnp.int32)
    unit_expert = jnp.sum(
        (u[:, None] >= tile_off[None, 1:]).astype(jnp.int32), axis=1
    )
    unit_expert = jnp.minimum(unit_expert, E - 1)
    return pos, src_ids, dst_ids, unit_expert


def kernel(x, Wred, wg, W1, b1, W2, b2):
    xf = x.reshape(T, C)
    scores, idx = _gating_tc(xf, Wred, wg)
    idx = idx[:, 0]
    pos, src_ids, dst_ids, unit_expert = _route(idx)
    scores_pad = jnp.zeros((NPAD, 1), jnp.float32).at[pos].set(scores)
    xs = _sc_gather(xf, src_ids)
    ys, total = _ffn_tc(xs, scores_pad, unit_expert, W1, b1, W2, b2)
    out = _sc_scatter(ys, dst_ids)[:T]
    return (out, total[0, 0])
